# baseline (speedup 1.0000x reference)
import jax
import jax.numpy as jnp
from jax.experimental import pallas as pl

BLOCK_ROWS = 4096


def _fwd(p_ref, wt_ref, b_ref, o_ref):
    p = p_ref[...]
    logits = jnp.dot(p, wt_ref[...], preferred_element_type=jnp.float32) + b_ref[...]
    m = jnp.max(logits, axis=-1, keepdims=True)
    e = jnp.exp(logits - m)
    o_ref[...] = e / jnp.sum(e, axis=-1, keepdims=True)


def kernel(payload, aux, W, b):
    n_tokens, token_dim = payload.shape
    out_dim = W.shape[0]
    wt = W.T
    b2 = b.reshape(1, out_dim)
    grid = (n_tokens // BLOCK_ROWS,)
    return pl.pallas_call(
        _fwd,
        grid=grid,
        in_specs=[
            pl.BlockSpec((BLOCK_ROWS, token_dim), lambda i: (i, 0)),
            pl.BlockSpec((token_dim, out_dim), lambda i: (0, 0)),
            pl.BlockSpec((1, out_dim), lambda i: (0, 0)),
        ],
        out_specs=pl.BlockSpec((BLOCK_ROWS, out_dim), lambda i: (i, 0)),
        out_shape=jax.ShapeDtypeStruct((n_tokens, out_dim), jnp.float32),
    )(payload, wt, b2)


# speedup vs baseline: 1.0888x; 1.0888x over previous
import jax
import jax.numpy as jnp
from jax.experimental import pallas as pl
from jax.experimental.pallas import tpu as pltpu

BLOCK_ROWS = 8192


def _fwd(p_ref, wt_ref, b_ref, o_ref):
    p = p_ref[...]
    logits = jnp.dot(p, wt_ref[...], preferred_element_type=jnp.float32) + b_ref[...]
    m = jnp.max(logits, axis=-1, keepdims=True)
    e = jnp.exp(logits - m)
    o_ref[...] = e / jnp.sum(e, axis=-1, keepdims=True)


def kernel(payload, aux, W, b):
    n_tokens, token_dim = payload.shape
    out_dim = W.shape[0]
    wt = W.T
    b2 = b.reshape(1, out_dim)
    grid = (n_tokens // BLOCK_ROWS,)
    return pl.pallas_call(
        _fwd,
        grid=grid,
        in_specs=[
            pl.BlockSpec((BLOCK_ROWS, token_dim), lambda i: (i, 0)),
            pl.BlockSpec((token_dim, out_dim), lambda i: (0, 0)),
            pl.BlockSpec((1, out_dim), lambda i: (0, 0)),
        ],
        out_specs=pl.BlockSpec((BLOCK_ROWS, out_dim), lambda i: (i, 0)),
        out_shape=jax.ShapeDtypeStruct((n_tokens, out_dim), jnp.float32),
        compiler_params=pltpu.CompilerParams(
            dimension_semantics=("parallel",),
        ),
    )(payload, wt, b2)
